# Initial kernel scaffold; baseline (speedup 1.0000x reference)
import jax
import jax.numpy as jnp
from jax.experimental import pallas as pl


def kernel(x, table, pe):
    raise NotImplementedError("write your pallas kernel here")


# speedup vs baseline: 10.9943x; 10.9943x over previous
import functools
import jax
import jax.numpy as jnp
from jax import lax
from jax.experimental import pallas as pl
from jax.experimental.pallas import tpu as pltpu
from jax.experimental.pallas import tpu_sc as plsc

N_TOKENS = 8192
D = 128
B = 64 * 2048
NC, NS = 2, 16
NW = NC * NS
B_PER_W = B // NW
CHUNK = 128
N_CHUNKS = B_PER_W // CHUNK


def _fuse_body(t_ref, p_ref, o_ref):
    o_ref[...] = t_ref[...] + p_ref[...]


def _fuse_tables(table, pe):
    blk = 1024
    return pl.pallas_call(
        _fuse_body,
        grid=(N_TOKENS // blk,),
        in_specs=[
            pl.BlockSpec((blk, D), lambda i: (i, 0)),
            pl.BlockSpec((blk, D), lambda i: (i, 0)),
        ],
        out_specs=pl.BlockSpec((blk, D), lambda i: (i, 0)),
        out_shape=jax.ShapeDtypeStruct((N_TOKENS, D), jnp.float32),
    )(table, pe)


@functools.partial(
    pl.kernel,
    out_type=jax.ShapeDtypeStruct((B, D), jnp.float32),
    mesh=plsc.VectorSubcoreMesh(core_axis_name="c", subcore_axis_name="s"),
    scratch_types=[
        pltpu.VMEM((N_CHUNKS, CHUNK), jnp.int32),
        pltpu.VMEM((2, CHUNK, D), jnp.float32),
        pltpu.SemaphoreType.DMA,
        pltpu.SemaphoreType.DMA,
    ],
)
def _sc_gather(idx_hbm, fused_hbm, out_hbm, idx_v, rows_v, gsem0, gsem1):
    wid = lax.axis_index("s") * NC + lax.axis_index("c")
    base = wid * B_PER_W
    pltpu.sync_copy(idx_hbm.at[wid], idx_v)

    def pair(g, carry):
        c0 = 2 * g
        c1 = c0 + 1
        cp0 = pltpu.async_copy(fused_hbm.at[idx_v.at[c0]], rows_v.at[0], gsem0)
        cp1 = pltpu.async_copy(fused_hbm.at[idx_v.at[c1]], rows_v.at[1], gsem1)
        cp0.wait()
        pltpu.sync_copy(rows_v.at[0], out_hbm.at[pl.ds(base + c0 * CHUNK, CHUNK)])
        cp1.wait()
        pltpu.sync_copy(rows_v.at[1], out_hbm.at[pl.ds(base + c1 * CHUNK, CHUNK)])
        return carry

    lax.fori_loop(0, N_CHUNKS // 2, pair, 0)


def kernel(x, table, pe):
    fused = _fuse_tables(table, pe)
    idx = x.reshape(NW, N_CHUNKS, CHUNK).astype(jnp.int32)
    out = _sc_gather(idx, fused)
    return out.reshape(x.shape[0], x.shape[1], D)


# speedup vs baseline: 11.2920x; 1.0271x over previous
import functools
import jax
import jax.numpy as jnp
from jax import lax
from jax.experimental import pallas as pl
from jax.experimental.pallas import tpu as pltpu
from jax.experimental.pallas import tpu_sc as plsc

N_TOKENS = 8192
D = 128
B = 64 * 2048
NC, NS = 2, 16
NW = NC * NS
B_PER_W = B // NW
CHUNK = 128
N_CHUNKS = B_PER_W // CHUNK


def _fuse_body(t_ref, p_ref, o_ref):
    o_ref[...] = t_ref[...] + p_ref[...]


def _fuse_tables(table, pe):
    blk = 1024
    return pl.pallas_call(
        _fuse_body,
        grid=(N_TOKENS // blk,),
        in_specs=[
            pl.BlockSpec((blk, D), lambda i: (i, 0)),
            pl.BlockSpec((blk, D), lambda i: (i, 0)),
        ],
        out_specs=pl.BlockSpec((blk, D), lambda i: (i, 0)),
        out_shape=jax.ShapeDtypeStruct((N_TOKENS, D), jnp.float32),
    )(table, pe)


@functools.partial(
    pl.kernel,
    out_type=jax.ShapeDtypeStruct((B, D), jnp.float32),
    mesh=plsc.VectorSubcoreMesh(core_axis_name="c", subcore_axis_name="s"),
    scratch_types=[
        pltpu.VMEM((N_CHUNKS, CHUNK), jnp.int32),
        pltpu.VMEM((2, CHUNK, D), jnp.float32),
        pltpu.SemaphoreType.DMA,
        pltpu.SemaphoreType.DMA,
        pltpu.SemaphoreType.DMA,
        pltpu.SemaphoreType.DMA,
    ],
)
def _sc_gather(idx_hbm, fused_hbm, out_hbm, idx_v, rows_v,
               gsem0, gsem1, wsem0, wsem1):
    wid = lax.axis_index("s") * NC + lax.axis_index("c")
    base = wid * B_PER_W
    pltpu.sync_copy(idx_hbm.at[wid], idx_v)

    def gather(c, buf, sem):
        pltpu.async_copy(fused_hbm.at[idx_v.at[c]], rows_v.at[buf], sem)

    def write(c, buf, sem):
        pltpu.async_copy(
            rows_v.at[buf], out_hbm.at[pl.ds(base + c * CHUNK, CHUNK)], sem)

    def wait_gather(buf, sem):
        pltpu.make_async_copy(
            fused_hbm.at[idx_v.at[0]], rows_v.at[buf], sem).wait()

    def wait_write(buf, sem):
        pltpu.make_async_copy(
            rows_v.at[buf], out_hbm.at[pl.ds(base, CHUNK)], sem).wait()

    gather(0, 0, gsem0)
    gather(1, 1, gsem1)
    wait_gather(0, gsem0)
    write(0, 0, wsem0)
    wait_gather(1, gsem1)
    write(1, 1, wsem1)

    def pair(g, carry):
        c0 = 2 * g
        c1 = c0 + 1
        wait_write(0, wsem0)
        gather(c0, 0, gsem0)
        wait_write(1, wsem1)
        gather(c1, 1, gsem1)
        wait_gather(0, gsem0)
        write(c0, 0, wsem0)
        wait_gather(1, gsem1)
        write(c1, 1, wsem1)
        return carry

    lax.fori_loop(1, N_CHUNKS // 2, pair, 0)
    wait_write(0, wsem0)
    wait_write(1, wsem1)


def kernel(x, table, pe):
    fused = _fuse_tables(table, pe)
    idx = x.reshape(NW, N_CHUNKS, CHUNK).astype(jnp.int32)
    out = _sc_gather(idx, fused)
    return out.reshape(x.shape[0], x.shape[1], D)


# speedup vs baseline: 11.5107x; 1.0194x over previous
import functools
import jax
import jax.numpy as jnp
from jax import lax
from jax.experimental import pallas as pl
from jax.experimental.pallas import tpu as pltpu
from jax.experimental.pallas import tpu_sc as plsc

N_TOKENS = 8192
D = 128
B = 64 * 2048
NC, NS = 2, 16
NW = NC * NS
B_PER_W = B // NW
CHUNK = 128
N_CHUNKS = B_PER_W // CHUNK


def _fuse_body(t_ref, p_ref, o_ref):
    o_ref[...] = t_ref[...] + p_ref[...]


def _fuse_tables(table, pe):
    blk = 1024
    return pl.pallas_call(
        _fuse_body,
        grid=(N_TOKENS // blk,),
        in_specs=[
            pl.BlockSpec((blk, D), lambda i: (i, 0)),
            pl.BlockSpec((blk, D), lambda i: (i, 0)),
        ],
        out_specs=pl.BlockSpec((blk, D), lambda i: (i, 0)),
        out_shape=jax.ShapeDtypeStruct((N_TOKENS, D), jnp.float32),
    )(table, pe)


@functools.partial(
    pl.kernel,
    out_type=jax.ShapeDtypeStruct((B, D), jnp.float32),
    mesh=plsc.VectorSubcoreMesh(core_axis_name="c", subcore_axis_name="s"),
    scratch_types=[
        pltpu.VMEM((N_CHUNKS, CHUNK), jnp.int32),
        pltpu.VMEM((4, CHUNK, D), jnp.float32),
        pltpu.SemaphoreType.DMA,
        pltpu.SemaphoreType.DMA,
        pltpu.SemaphoreType.DMA,
        pltpu.SemaphoreType.DMA,
        pltpu.SemaphoreType.DMA,
        pltpu.SemaphoreType.DMA,
        pltpu.SemaphoreType.DMA,
        pltpu.SemaphoreType.DMA,
    ],
)
def _sc_gather(idx_hbm, fused_hbm, out_hbm, idx_v, rows_v,
               gsem0, gsem1, gsem2, gsem3, wsem0, wsem1, wsem2, wsem3):
    wid = lax.axis_index("s") * NC + lax.axis_index("c")
    base = wid * B_PER_W
    gsems = (gsem0, gsem1, gsem2, gsem3)
    wsems = (wsem0, wsem1, wsem2, wsem3)
    pltpu.sync_copy(idx_hbm.at[wid], idx_v)

    def gather(c, buf):
        pltpu.async_copy(fused_hbm.at[idx_v.at[c]], rows_v.at[buf], gsems[buf])

    def write(c, buf):
        pltpu.async_copy(
            rows_v.at[buf], out_hbm.at[pl.ds(base + c * CHUNK, CHUNK)],
            wsems[buf])

    def wait_gather(buf):
        pltpu.make_async_copy(
            fused_hbm.at[idx_v.at[0]], rows_v.at[buf], gsems[buf]).wait()

    def wait_write(buf):
        pltpu.make_async_copy(
            rows_v.at[buf], out_hbm.at[pl.ds(base, CHUNK)], wsems[buf]).wait()

    for b in range(4):
        gather(b, b)
    for b in range(4):
        wait_gather(b)
        write(b, b)

    def group(g, carry):
        c = 4 * g
        for b in range(4):
            wait_write(b)
            gather(c + b, b)
        for b in range(4):
            wait_gather(b)
            write(c + b, b)
        return carry

    lax.fori_loop(1, N_CHUNKS // 4, group, 0)
    for b in range(4):
        wait_write(b)


def kernel(x, table, pe):
    fused = _fuse_tables(table, pe)
    idx = x.reshape(NW, N_CHUNKS, CHUNK).astype(jnp.int32)
    out = _sc_gather(idx, fused)
    return out.reshape(x.shape[0], x.shape[1], D)
